# baseline (speedup 1.0000x reference)
import functools

import jax
import jax.numpy as jnp
from jax import lax
from jax.experimental import pallas as pl
from jax.experimental.pallas import tpu as pltpu
from jax.experimental.pallas import tpu_sc as plsc

_N = 10000
_E = 320000
_D = 128
_LANES = 16
_N_PAD = 10240
_ROWS = 5120
_NS = 16
_CHUNK2 = _E // _NS
_WIN2 = 20096
_HALF = 9984


def _sc_flags(eic, eib):
  mesh = plsc.VectorSubcoreMesh(core_axis_name="c", subcore_axis_name="s")

  @functools.partial(
      pl.kernel,
      mesh=mesh,
      out_type=(
          jax.ShapeDtypeStruct((_NS, _N_PAD), jnp.float32),
          jax.ShapeDtypeStruct((_NS, _N_PAD), jnp.float32),
      ),
      scratch_types=[
          pltpu.VMEM((2, _WIN2), jnp.int32),
          pltpu.VMEM((_N_PAD,), jnp.float32),
          pltpu.SemaphoreType.DMA,
          pltpu.SemaphoreType.DMA,
      ],
      compiler_params=pltpu.CompilerParams(needs_layout_passes=False),
  )
  def k(eic_hbm, eib_hbm, fc_hbm, fb_hbm, idx_v, flag_v, sem, sem2):
    core = lax.axis_index("c")
    tid = lax.axis_index("s")
    base = tid * _CHUNK2
    s = lax.rem(base, 128)
    astart = pl.multiple_of(base - s, 128)
    ones = jnp.ones((_LANES,), jnp.float32)
    zeros = jnp.zeros((_LANES,), jnp.float32)

    def one_etype(e_hbm, f_hbm):
      cp0 = pltpu.async_copy(
          e_hbm.at[:, pl.ds(astart, _HALF)], idx_v.at[:, pl.ds(0, _HALF)],
          sem)
      cp1 = pltpu.async_copy(
          e_hbm.at[:, pl.ds(astart + _HALF, _WIN2 - _HALF)],
          idx_v.at[:, pl.ds(_HALF, _WIN2 - _HALF)], sem2)

      def zero_body(i, carry):
        for j in range(4):
          flag_v[pl.ds((i * 4 + j) * _LANES, _LANES)] = zeros
        return carry

      lax.fori_loop(0, _N_PAD // _LANES // 4, zero_body, 0)

      unroll = 4

      def body(i, carry):
        for j in range(unroll):
          o = pl.ds((i * unroll + j) * _LANES, _LANES)
          plsc.store_scatter(flag_v, [idx_v[1, o]], ones)
        return carry

      cp0.wait()
      lax.fori_loop(0, _HALF // _LANES // unroll, body, 0)
      cp1.wait()
      lax.fori_loop(_HALF // _LANES // unroll, _WIN2 // _LANES // unroll,
                    body, 0)
      pltpu.sync_copy(flag_v, f_hbm.at[tid])

    @pl.when(core == 0)
    def _():
      one_etype(eic_hbm, fc_hbm)

    @pl.when(core == 1)
    def _():
      one_etype(eib_hbm, fb_hbm)

  return k(eic, eib)


def _tc_body(fu, fi, eu, ei, wc, bc, wb, bb, wu, bu, wi, bi, ou, oi):
  v_mc = jnp.maximum(eu[...] @ wc[...] + bc[...], 0.0)
  v_mb = jnp.maximum(ei[...] @ wb[...] + bb[...], 0.0)
  base_u = eu[...] @ wu[:_D] + bu[...]
  row_a_u = jnp.maximum(base_u + v_mb @ wu[_D:], 0.0)
  row_b_u = jnp.maximum(base_u, 0.0)
  base_i = ei[...] @ wi[:_D] + bi[...]
  row_a_i = jnp.maximum(base_i + v_mc @ wi[_D:], 0.0)
  row_b_i = jnp.maximum(base_i, 0.0)
  fu_blk = jnp.max(fu[...], axis=0)
  fi_blk = jnp.max(fi[...], axis=0)
  ou[...] = jnp.where(fu_blk[:, None] > 0.0, row_a_u, row_b_u)
  oi[...] = jnp.where(fi_blk[:, None] > 0.0, row_a_i, row_b_i)


def _tc_assemble(flags_u, flags_i, emb_u, emb_i, we_c, be_c, we_b, be_b,
                 wn_u, bn_u, wn_i, bn_i):
  full = lambda s: pl.BlockSpec(s, lambda j: (0,) * len(s))
  return pl.pallas_call(
      _tc_body,
      grid=(_N_PAD // _ROWS,),
      in_specs=[
          pl.BlockSpec((_NS, _ROWS), lambda j: (0, j)),
          pl.BlockSpec((_NS, _ROWS), lambda j: (0, j)),
          full((1, _D)), full((1, _D)),
          full((_D, _D)), full((1, _D)),
          full((_D, _D)), full((1, _D)),
          full((2 * _D, _D)), full((1, _D)),
          full((2 * _D, _D)), full((1, _D)),
      ],
      out_specs=[
          pl.BlockSpec((_ROWS, _D), lambda j: (j, 0)),
          pl.BlockSpec((_ROWS, _D), lambda j: (j, 0)),
      ],
      out_shape=[jax.ShapeDtypeStruct((_N, _D), jnp.float32)] * 2,
  )(flags_u, flags_i, emb_u, emb_i,
    we_c, be_c.reshape(1, _D), we_b, be_b.reshape(1, _D),
    wn_u, bn_u.reshape(1, _D), wn_i, bn_i.reshape(1, _D))


def kernel(edge_index_clicks, edge_index_clicked_by, emb_user, emb_item,
           We_clicks, be_clicks, We_cb, be_cb,
           Wn_user, bn_user, Wn_item, bn_item):
  eic = edge_index_clicks.astype(jnp.int32)
  eib = edge_index_clicked_by.astype(jnp.int32)
  flags_item, flags_user = _sc_flags(eic, eib)
  out_u, out_i = _tc_assemble(
      flags_user, flags_item, emb_user, emb_item,
      We_clicks, be_clicks, We_cb, be_cb,
      Wn_user, bn_user, Wn_item, bn_item)
  return out_u, out_i


# speedup vs baseline: 1.0076x; 1.0076x over previous
import functools

import jax
import jax.numpy as jnp
from jax import lax
from jax.experimental import pallas as pl
from jax.experimental.pallas import tpu as pltpu
from jax.experimental.pallas import tpu_sc as plsc

_N = 10000
_E = 320000
_D = 128
_LANES = 16
_N_PAD = 10240
_ROWS = 5120
_NS = 16
_CHUNK2 = _E // _NS
_WIN2 = 20096


def _sc_flags(eic, eib):
  mesh = plsc.VectorSubcoreMesh(core_axis_name="c", subcore_axis_name="s")

  @functools.partial(
      pl.kernel,
      mesh=mesh,
      out_type=(
          jax.ShapeDtypeStruct((_NS, _N_PAD), jnp.float32),
          jax.ShapeDtypeStruct((_NS, _N_PAD), jnp.float32),
      ),
      scratch_types=[
          pltpu.VMEM((2, _WIN2), jnp.int32),
          pltpu.VMEM((_N_PAD,), jnp.float32),
          pltpu.SemaphoreType.DMA,
      ],
      compiler_params=pltpu.CompilerParams(needs_layout_passes=False),
  )
  def k(eic_hbm, eib_hbm, fc_hbm, fb_hbm, idx_v, flag_v, sem):
    core = lax.axis_index("c")
    tid = lax.axis_index("s")
    base = tid * _CHUNK2
    s = lax.rem(base, 128)
    astart = pl.multiple_of(base - s, 128)
    ones = jnp.ones((_LANES,), jnp.float32)
    zeros = jnp.zeros((_LANES,), jnp.float32)

    def one_etype(e_hbm, f_hbm):
      cp = pltpu.async_copy(e_hbm.at[:, pl.ds(astart, _WIN2)], idx_v, sem)

      def zero_body(i, carry):
        for j in range(4):
          flag_v[pl.ds((i * 4 + j) * _LANES, _LANES)] = zeros
        return carry

      lax.fori_loop(0, _N_PAD // _LANES // 4, zero_body, 0)
      cp.wait()

      unroll = 4
      n_outer = _CHUNK2 // _LANES // unroll

      def body(i, carry):
        for j in range(unroll):
          o = pl.ds(s + (i * unroll + j) * _LANES, _LANES)
          plsc.store_scatter(flag_v, [idx_v[1, o]], ones)
        return carry

      lax.fori_loop(0, n_outer, body, 0)
      pltpu.sync_copy(flag_v, f_hbm.at[tid])

    @pl.when(core == 0)
    def _():
      one_etype(eic_hbm, fc_hbm)

    @pl.when(core == 1)
    def _():
      one_etype(eib_hbm, fb_hbm)

  return k(eic, eib)


def _tc_body(fu, fi, eu, ei, wc, bc, wb, bb, wu, bu, wi, bi, ou, oi):
  v_mc = jnp.maximum(eu[...] @ wc[...] + bc[...], 0.0)
  v_mb = jnp.maximum(ei[...] @ wb[...] + bb[...], 0.0)
  base_u = eu[...] @ wu[:_D] + bu[...]
  row_a_u = jnp.maximum(base_u + v_mb @ wu[_D:], 0.0)
  row_b_u = jnp.maximum(base_u, 0.0)
  base_i = ei[...] @ wi[:_D] + bi[...]
  row_a_i = jnp.maximum(base_i + v_mc @ wi[_D:], 0.0)
  row_b_i = jnp.maximum(base_i, 0.0)
  fu_blk = jnp.max(fu[...], axis=0)
  fi_blk = jnp.max(fi[...], axis=0)
  ou[...] = jnp.where(fu_blk[:, None] > 0.0, row_a_u, row_b_u)
  oi[...] = jnp.where(fi_blk[:, None] > 0.0, row_a_i, row_b_i)


def _tc_assemble(flags_u, flags_i, emb_u, emb_i, we_c, be_c, we_b, be_b,
                 wn_u, bn_u, wn_i, bn_i):
  full = lambda s: pl.BlockSpec(s, lambda j: (0,) * len(s))
  return pl.pallas_call(
      _tc_body,
      grid=(_N_PAD // _ROWS,),
      in_specs=[
          pl.BlockSpec((_NS, _ROWS), lambda j: (0, j)),
          pl.BlockSpec((_NS, _ROWS), lambda j: (0, j)),
          full((1, _D)), full((1, _D)),
          full((_D, _D)), full((1, _D)),
          full((_D, _D)), full((1, _D)),
          full((2 * _D, _D)), full((1, _D)),
          full((2 * _D, _D)), full((1, _D)),
      ],
      out_specs=[
          pl.BlockSpec((_ROWS, _D), lambda j: (j, 0)),
          pl.BlockSpec((_ROWS, _D), lambda j: (j, 0)),
      ],
      out_shape=[jax.ShapeDtypeStruct((_N, _D), jnp.float32)] * 2,
  )(flags_u, flags_i, emb_u, emb_i,
    we_c, be_c.reshape(1, _D), we_b, be_b.reshape(1, _D),
    wn_u, bn_u.reshape(1, _D), wn_i, bn_i.reshape(1, _D))


def kernel(edge_index_clicks, edge_index_clicked_by, emb_user, emb_item,
           We_clicks, be_clicks, We_cb, be_cb,
           Wn_user, bn_user, Wn_item, bn_item):
  eic = edge_index_clicks.astype(jnp.int32)
  eib = edge_index_clicked_by.astype(jnp.int32)
  flags_item, flags_user = _sc_flags(eic, eib)
  out_u, out_i = _tc_assemble(
      flags_user, flags_item, emb_user, emb_item,
      We_clicks, be_clicks, We_cb, be_cb,
      Wn_user, bn_user, Wn_item, bn_item)
  return out_u, out_i
